# Initial kernel scaffold; baseline (speedup 1.0000x reference)
import jax
import jax.numpy as jnp
from jax.experimental import pallas as pl


def kernel(rank, X_in, ones):
    raise NotImplementedError("write your pallas kernel here")


# speedup vs baseline: 80.4829x; 80.4829x over previous
import jax
import jax.numpy as jnp
from jax.experimental import pallas as pl

DEPTH = 32
HB = 64


def _onehot_block(x_ref, out_ref):
    x = x_ref[...]
    d = jax.lax.broadcasted_iota(jnp.int32, (1, DEPTH, HB, 512), 1)
    out_ref[...] = (d == x).astype(jnp.float32)


def kernel(rank, X_in, ones):
    B, _, H, W = X_in.shape
    grid = (B, H // HB)
    out = pl.pallas_call(
        _onehot_block,
        grid=grid,
        in_specs=[pl.BlockSpec((1, 1, HB, W), lambda b, h: (b, 0, h, 0))],
        out_specs=pl.BlockSpec((1, DEPTH, HB, W), lambda b, h: (b, 0, h, 0)),
        out_shape=jax.ShapeDtypeStruct((B, DEPTH, H, W), jnp.float32),
    )(X_in)
    return out
